# baseline (speedup 1.0000x reference)
import functools

import jax
import jax.numpy as jnp
from jax import lax
from jax.experimental import pallas as pl
from jax.experimental.pallas import tpu as pltpu
from jax.experimental.pallas import tpu_sc as plsc

B = 128
N = 32768
LANES = 16
NUM_WORKERS = 32
ROWS_PER_W = B // NUM_WORKERS
U = 8
ITERS = N // (LANES * U)
BLKW = 512


def _row_argmax(buf):
    lane = lax.broadcasted_iota(jnp.int32, (LANES,), 0)
    neg_inf = jnp.full((LANES,), -jnp.inf, jnp.float32)
    zero_i = jnp.zeros((LANES,), jnp.int32)

    def body(i, carry):
        ivec, maxs, iters = carry
        maxs, iters = list(maxs), list(iters)
        base = i * (U * LANES)
        for u in range(U):
            v = buf[pl.ds(base + u * LANES, LANES)]
            pred = v > maxs[u]
            maxs[u] = jnp.where(pred, v, maxs[u])
            iters[u] = jnp.where(pred, ivec, iters[u])
        return ivec + 1, tuple(maxs), tuple(iters)

    _, maxs, iters = lax.fori_loop(
        0, ITERS, body, (zero_i, (neg_inf,) * U, (zero_i,) * U))

    best_m = maxs[0]
    best_p = (iters[0] * U + 0) * LANES + lane
    for u in range(1, U):
        p = (iters[u] * U + u) * LANES + lane
        better = (maxs[u] > best_m) | ((maxs[u] == best_m) & (p < best_p))
        best_m = jnp.where(better, maxs[u], best_m)
        best_p = jnp.where(better, p, best_p)
    m = jnp.max(best_m)
    cand = jnp.where(best_m == m, best_p, jnp.int32(2**30))
    return jnp.min(cand)


@functools.partial(
    pl.kernel,
    out_type=jax.ShapeDtypeStruct((NUM_WORKERS * 8,), jnp.int32),
    mesh=plsc.VectorSubcoreMesh(core_axis_name="c", subcore_axis_name="s"),
    compiler_params=pltpu.CompilerParams(needs_layout_passes=False),
    scratch_types=[
        pltpu.VMEM((N,), jnp.float32),
        pltpu.VMEM((N,), jnp.float32),
        pltpu.VMEM((LANES,), jnp.int32),
        pltpu.SemaphoreType.DMA((2,)),
        pltpu.SemaphoreType.DMA,
    ],
)
def _sc_argmax(x_hbm, pos_hbm, buf0, buf1, posbuf, sem_in, sem_out):
    wid = lax.axis_index("s") * 2 + lax.axis_index("c")
    row0 = wid * ROWS_PER_W

    bufs = [buf0, buf1]
    handles = [None] * ROWS_PER_W
    handles[0] = pltpu.async_copy(x_hbm.at[row0], bufs[0], sem_in.at[0])

    lane = lax.broadcasted_iota(jnp.int32, (LANES,), 0)
    posvec = jnp.zeros((LANES,), jnp.int32)
    for r in range(ROWS_PER_W):
        handles[r].wait()
        if r + 1 < ROWS_PER_W:
            handles[r + 1] = pltpu.async_copy(x_hbm.at[row0 + r + 1],
                                              bufs[(r + 1) % 2],
                                              sem_in.at[(r + 1) % 2])
        pos = _row_argmax(bufs[r % 2])
        posvec = jnp.where(lane == r, pos, posvec)

    posbuf[...] = posvec
    pltpu.async_copy(posbuf.at[pl.ds(0, 8)], pos_hbm.at[pl.ds(wid * 8, 8)],
                     sem_out).wait()


def _zeros_body(o_ref):
    o_ref[...] = jnp.zeros_like(o_ref)


_tc_zeros = pl.pallas_call(
    _zeros_body,
    grid=(8,),
    out_specs=pl.BlockSpec((B // 8, N), lambda i: (i, 0)),
    out_shape=jax.ShapeDtypeStruct((B, N), jnp.float32),
)


def _fix_body(blk_ref, win_ref, z_ref, o_ref):
    del z_ref
    r = pl.program_id(0)
    rb = (r // 8) * 8
    cur = blk_ref[r]
    col = lax.broadcasted_iota(jnp.int32, (8, BLKW), 1)
    row = lax.broadcasted_iota(jnp.int32, (8, BLKW), 0)
    acc = jnp.zeros((8, BLKW), jnp.float32)
    for j in range(8):
        hit = (blk_ref[rb + j] == cur) & (row == j) & (col == win_ref[rb + j])
        acc = jnp.where(hit, 1.0, acc)
    o_ref[...] = acc


_tc_fix = pl.pallas_call(
    _fix_body,
    grid_spec=pltpu.PrefetchScalarGridSpec(
        num_scalar_prefetch=2,
        grid=(B,),
        in_specs=[
            pl.BlockSpec((8, BLKW), lambda r, blk, win: (r // 8, blk[r]))
        ],
        out_specs=pl.BlockSpec((8, BLKW), lambda r, blk, win: (r // 8, blk[r])),
    ),
    out_shape=jax.ShapeDtypeStruct((B, N), jnp.float32),
    input_output_aliases={2: 0},
)


def kernel(x):
    pos8 = _sc_argmax(x)
    pos = pos8.reshape(NUM_WORKERS, 8)[:, :ROWS_PER_W].reshape(B)
    zeros = _tc_zeros()
    return _tc_fix(pos // BLKW, pos % BLKW, zeros)


# speedup vs baseline: 2.6305x; 2.6305x over previous
import functools

import jax
import jax.numpy as jnp
from jax import lax
from jax.experimental import pallas as pl
from jax.experimental.pallas import tpu as pltpu
from jax.experimental.pallas import tpu_sc as plsc

B = 128
N = 32768
LANES = 16
NUM_WORKERS = 32
ROWS_PER_W = B // NUM_WORKERS
U = 8
ITERS = N // (LANES * U)
BLKW = 512


def _row_argmax(buf):
    lane = lax.broadcasted_iota(jnp.int32, (LANES,), 0)
    neg_inf = jnp.full((LANES,), -jnp.inf, jnp.float32)
    zero_i = jnp.zeros((LANES,), jnp.int32)

    def body(i, carry):
        ivec, maxs, iters = carry
        maxs, iters = list(maxs), list(iters)
        base = i * (U * LANES)
        for u in range(U):
            v = buf[pl.ds(base + u * LANES, LANES)]
            pred = v > maxs[u]
            maxs[u] = jnp.where(pred, v, maxs[u])
            iters[u] = jnp.where(pred, ivec, iters[u])
        return ivec + 1, tuple(maxs), tuple(iters)

    _, maxs, iters = lax.fori_loop(
        0, ITERS, body, (zero_i, (neg_inf,) * U, (zero_i,) * U))

    best_m = maxs[0]
    best_p = (iters[0] * U + 0) * LANES + lane
    for u in range(1, U):
        p = (iters[u] * U + u) * LANES + lane
        better = (maxs[u] > best_m) | ((maxs[u] == best_m) & (p < best_p))
        best_m = jnp.where(better, maxs[u], best_m)
        best_p = jnp.where(better, p, best_p)
    m = jnp.max(best_m)
    cand = jnp.where(best_m == m, best_p, jnp.int32(2**30))
    return jnp.min(cand)


@functools.partial(
    pl.kernel,
    out_type=jax.ShapeDtypeStruct((NUM_WORKERS * 8,), jnp.int32),
    mesh=plsc.VectorSubcoreMesh(core_axis_name="c", subcore_axis_name="s"),
    compiler_params=pltpu.CompilerParams(needs_layout_passes=False),
    scratch_types=[
        pltpu.VMEM((N,), jnp.float32),
        pltpu.VMEM((N,), jnp.float32),
        pltpu.VMEM((LANES,), jnp.int32),
        pltpu.SemaphoreType.DMA((2,)),
        pltpu.SemaphoreType.DMA,
    ],
)
def _sc_argmax(x_hbm, pos_hbm, buf0, buf1, posbuf, sem_in, sem_out):
    wid = lax.axis_index("s") * 2 + lax.axis_index("c")
    row0 = wid * ROWS_PER_W

    bufs = [buf0, buf1]
    handles = [None] * ROWS_PER_W
    handles[0] = pltpu.async_copy(x_hbm.at[row0], bufs[0], sem_in.at[0])

    lane = lax.broadcasted_iota(jnp.int32, (LANES,), 0)
    posvec = jnp.zeros((LANES,), jnp.int32)
    for r in range(ROWS_PER_W):
        handles[r].wait()
        if r + 1 < ROWS_PER_W:
            handles[r + 1] = pltpu.async_copy(x_hbm.at[row0 + r + 1],
                                              bufs[(r + 1) % 2],
                                              sem_in.at[(r + 1) % 2])
        pos = _row_argmax(bufs[r % 2])
        posvec = jnp.where(lane == r, pos, posvec)

    posbuf[...] = posvec
    pltpu.async_copy(posbuf.at[pl.ds(0, 8)], pos_hbm.at[pl.ds(wid * 8, 8)],
                     sem_out).wait()


def _zeros_body(o_ref):
    o_ref[...] = jnp.zeros_like(o_ref)


_tc_zeros = pl.pallas_call(
    _zeros_body,
    grid=(8,),
    out_specs=pl.BlockSpec((B // 8, N), lambda i: (i, 0)),
    out_shape=jax.ShapeDtypeStruct((B, N), jnp.float32),
)


def _fix_body(pos_smem, eye_ref, z_ref, o_ref, sem):
    del z_ref

    def issue(r, carry):
        widx = (r // ROWS_PER_W) * 8 + r % ROWS_PER_W
        p = pos_smem[widx]
        base = (p // 128) * 128
        win = p - base
        pltpu.make_async_copy(
            eye_ref.at[pl.ds(win, 1), :],
            o_ref.at[pl.ds(r, 1), pl.ds(base, 128)], sem).start()
        return carry

    lax.fori_loop(0, B, issue, 0)
    pltpu.make_async_copy(eye_ref, o_ref.at[:, pl.ds(0, 128)], sem).wait()


_tc_fix = pl.pallas_call(
    _fix_body,
    grid_spec=pltpu.PrefetchScalarGridSpec(
        num_scalar_prefetch=1,
        grid=(1,),
        in_specs=[
            pl.BlockSpec((128, 128), lambda i, pos: (0, 0)),
            pl.BlockSpec(memory_space=pltpu.MemorySpace.HBM),
        ],
        out_specs=pl.BlockSpec(memory_space=pltpu.MemorySpace.HBM),
        scratch_shapes=[pltpu.SemaphoreType.DMA],
    ),
    out_shape=jax.ShapeDtypeStruct((B, N), jnp.float32),
    input_output_aliases={2: 0},
)


def kernel(x):
    pos8 = _sc_argmax(x)
    zeros = _tc_zeros()
    eye = jnp.eye(128, dtype=jnp.float32)
    return _tc_fix(pos8, eye, zeros)
